# speedup vs baseline: 1.5484x; 1.5484x over previous
import functools

import jax
import jax.numpy as jnp
from jax import lax
from jax.experimental import pallas as pl
from jax.experimental.pallas import tpu as pltpu
from jax.experimental.pallas import tpu_sc as plsc

_NUM_CORES = 2
_NUM_SUBCORES = 16
_NUM_WORKERS = _NUM_CORES * _NUM_SUBCORES

_CHUNK = 64


def _emb_lookup(idx, table, n_per_w, steps, hidden):
    mesh = plsc.VectorSubcoreMesh(core_axis_name="c", subcore_axis_name="s")
    n_total = idx.shape[0]

    @functools.partial(
        pl.kernel,
        out_type=jax.ShapeDtypeStruct((n_total, hidden), jnp.float32),
        mesh=mesh,
        scratch_types=[
            pltpu.VMEM((n_per_w,), jnp.int32),
            pltpu.VMEM((_CHUNK, hidden), jnp.float32),
            pltpu.SemaphoreType.DMA,
        ],
    )
    def emb(idx_hbm, table_hbm, out_hbm, idx_v, rows_v, gsem):
        wid = lax.axis_index("s") * _NUM_CORES + lax.axis_index("c")
        base = wid * n_per_w
        pltpu.sync_copy(idx_hbm.at[pl.ds(base, n_per_w)], idx_v)

        def step(s, carry):
            off = s * _CHUNK
            pltpu.async_copy(
                table_hbm.at[idx_v.at[pl.ds(off, _CHUNK)]], rows_v, gsem
            ).wait()
            pltpu.sync_copy(rows_v, out_hbm.at[pl.ds(base + off, _CHUNK)])
            return carry

        lax.fori_loop(0, steps, step, 0)

    return emb(idx, table)


def kernel(src, tgt, seg, word_table):
    b, s = src.shape
    _, hidden = word_table.shape
    n = b * s
    n_per_w = n // _NUM_WORKERS
    steps = n_per_w // _CHUNK
    idx = src.reshape(n).astype(jnp.int32)
    out = _emb_lookup(idx, word_table, n_per_w, steps, hidden)
    return (out.reshape(b, s, hidden), tgt, seg)


# speedup vs baseline: 1.5791x; 1.0198x over previous
import functools

import jax
import jax.numpy as jnp
from jax import lax
from jax.experimental import pallas as pl
from jax.experimental.pallas import tpu as pltpu
from jax.experimental.pallas import tpu_sc as plsc

_NUM_CORES = 2
_NUM_SUBCORES = 16
_NUM_WORKERS = _NUM_CORES * _NUM_SUBCORES

_CHUNK = 32


def _emb_lookup(idx, table, n_per_w, steps, hidden):
    mesh = plsc.VectorSubcoreMesh(core_axis_name="c", subcore_axis_name="s")
    n_total = idx.shape[0]
    assert steps >= 4 and steps % 2 == 0

    @functools.partial(
        pl.kernel,
        out_type=jax.ShapeDtypeStruct((n_total, hidden), jnp.float32),
        mesh=mesh,
        scratch_types=[
            pltpu.VMEM((n_per_w,), jnp.int32),
            pltpu.VMEM((_CHUNK, hidden), jnp.float32),
            pltpu.VMEM((_CHUNK, hidden), jnp.float32),
            pltpu.SemaphoreType.DMA,
            pltpu.SemaphoreType.DMA,
            pltpu.SemaphoreType.DMA,
            pltpu.SemaphoreType.DMA,
        ],
    )
    def emb(idx_hbm, table_hbm, out_hbm, idx_v, rows0, rows1, g0, g1, p0, p1):
        wid = lax.axis_index("s") * _NUM_CORES + lax.axis_index("c")
        base = wid * n_per_w
        pltpu.sync_copy(idx_hbm.at[pl.ds(base, n_per_w)], idx_v)

        def start_gather(s, buf, sem):
            pltpu.async_copy(table_hbm.at[idx_v.at[pl.ds(s * _CHUNK, _CHUNK)]], buf, sem)

        def start_put(s, buf, sem):
            pltpu.async_copy(buf, out_hbm.at[pl.ds(base + s * _CHUNK, _CHUNK)], sem)

        def wait_gather(buf, sem):
            pltpu.make_async_copy(table_hbm.at[pl.ds(0, _CHUNK)], buf, sem).wait()

        def wait_put(buf, sem):
            pltpu.make_async_copy(buf, out_hbm.at[pl.ds(0, _CHUNK)], sem).wait()

        start_gather(0, rows0, g0)
        wait_gather(rows0, g0)
        start_put(0, rows0, p0)
        start_gather(1, rows1, g1)

        def group(g, carry):
            s1 = 2 * g + 1
            wait_gather(rows1, g1)
            start_put(s1, rows1, p1)
            wait_put(rows0, p0)
            start_gather(s1 + 1, rows0, g0)

            s2 = 2 * g + 2
            wait_gather(rows0, g0)
            start_put(s2, rows0, p0)
            wait_put(rows1, p1)
            start_gather(s2 + 1, rows1, g1)
            return carry

        lax.fori_loop(0, (steps - 2) // 2, group, 0)

        wait_gather(rows1, g1)
        start_put(steps - 1, rows1, p1)
        wait_put(rows0, p0)
        wait_put(rows1, p1)

    return emb(idx, table)


def kernel(src, tgt, seg, word_table):
    b, s = src.shape
    _, hidden = word_table.shape
    n = b * s
    n_per_w = n // _NUM_WORKERS
    steps = n_per_w // _CHUNK
    idx = src.reshape(n).astype(jnp.int32)
    out = _emb_lookup(idx, word_table, n_per_w, steps, hidden)
    return (out.reshape(b, s, hidden), tgt, seg)
